# Initial kernel scaffold; baseline (speedup 1.0000x reference)
import jax
import jax.numpy as jnp
from jax.experimental import pallas as pl


def kernel(features, edge_index, W0, b0, W1, b1, W2, b2):
    raise NotImplementedError("write your pallas kernel here")


# speedup vs baseline: 7.0024x; 7.0024x over previous
import functools

import jax
import jax.numpy as jnp
from jax import lax
from jax.experimental import pallas as pl
from jax.experimental.pallas import tpu as pltpu
from jax.experimental.pallas import tpu_sc as plsc

_N = 10000
_E = 320000
_NC = 2
_NS = 16
_NW = _NC * _NS
_EPT = _E // _NW
_CHUNK = 80
_NCHUNK = _EPT // _CHUNK
_RPT = _N // _NS

_M_BLK = 2000


def _make_aggregate(d):
    mesh = plsc.VectorSubcoreMesh(core_axis_name="c", subcore_axis_name="s")

    @functools.partial(
        pl.kernel,
        out_type=jax.ShapeDtypeStruct((_NC, _NS, _RPT, d), jnp.float32),
        mesh=mesh,
        scratch_types=[
            pltpu.VMEM((_NCHUNK, _CHUNK), jnp.int32),
            pltpu.VMEM((_NCHUNK, _CHUNK), jnp.int32),
            pltpu.VMEM((_CHUNK, d), jnp.float32),
            pltpu.VMEM_SHARED((_N, d), jnp.float32),
            pltpu.SemaphoreType.DMA,
        ],
    )
    def agg(y_hbm, src_hbm, dst_hbm, zeros_hbm, out_hbm,
            src_v, dst_v, rows_v, accum, sem):
        cid = lax.axis_index("c")
        sid = lax.axis_index("s")
        slot = cid * _NS + sid

        pltpu.sync_copy(zeros_hbm, accum.at[pl.ds(sid * _RPT, _RPT)])
        pltpu.sync_copy(src_hbm.at[slot], src_v)
        pltpu.sync_copy(dst_hbm.at[slot], dst_v)
        plsc.subcore_barrier()

        def body(j, carry):
            pltpu.async_copy(y_hbm.at[src_v.at[j]], rows_v, sem).wait()
            pltpu.sync_copy(rows_v, accum.at[dst_v.at[j]], add=True)
            return carry

        lax.fori_loop(0, _NCHUNK, body, 0)
        plsc.subcore_barrier()
        pltpu.sync_copy(accum.at[pl.ds(sid * _RPT, _RPT)], out_hbm.at[cid, sid])

    return agg


_AGG128 = _make_aggregate(128)


def _mm_body(x_ref, w_ref, o_ref):
    o_ref[...] = jnp.dot(x_ref[...], w_ref[...],
                         preferred_element_type=jnp.float32)


def _matmul(x, w):
    m, k = x.shape
    n = w.shape[1]
    return pl.pallas_call(
        _mm_body,
        grid=(m // _M_BLK,),
        in_specs=[
            pl.BlockSpec((_M_BLK, k), lambda i: (i, 0)),
            pl.BlockSpec((k, n), lambda i: (0, 0)),
        ],
        out_specs=pl.BlockSpec((_M_BLK, n), lambda i: (i, 0)),
        out_shape=jax.ShapeDtypeStruct((m, n), jnp.float32),
    )(x, w)


def _fused_body(p_ref, b_ref, w_ref, o_ref):
    h = jnp.maximum(p_ref[0] + p_ref[1] + b_ref[...], 0.0)
    o_ref[...] = jnp.dot(h, w_ref[...], preferred_element_type=jnp.float32)


def _fused_matmul(p, b, w):
    _, m, k = p.shape
    n = w.shape[1]
    return pl.pallas_call(
        _fused_body,
        grid=(m // _M_BLK,),
        in_specs=[
            pl.BlockSpec((_NC, _M_BLK, k), lambda i: (0, i, 0)),
            pl.BlockSpec((1, k), lambda i: (0, 0)),
            pl.BlockSpec((k, n), lambda i: (0, 0)),
        ],
        out_specs=pl.BlockSpec((_M_BLK, n), lambda i: (i, 0)),
        out_shape=jax.ShapeDtypeStruct((m, n), jnp.float32),
    )(p, b, w)


def _relu_body(p_ref, b_ref, o_ref):
    o_ref[...] = jnp.maximum(p_ref[0] + p_ref[1] + b_ref[...], 0.0)


def _relu_add(p, b):
    _, m, n = p.shape
    return pl.pallas_call(
        _relu_body,
        grid=(m // _M_BLK,),
        in_specs=[
            pl.BlockSpec((_NC, _M_BLK, n), lambda i: (0, i, 0)),
            pl.BlockSpec((1, n), lambda i: (0, 0)),
        ],
        out_specs=pl.BlockSpec((_M_BLK, n), lambda i: (i, 0)),
        out_shape=jax.ShapeDtypeStruct((m, n), jnp.float32),
    )(p, b)


def _final_body(p_ref, w_ref, b_ref, o_ref):
    acc = jnp.dot(p_ref[0] + p_ref[1], w_ref[...],
                  preferred_element_type=jnp.float32)
    o_ref[...] = acc + b_ref[...]


def _final_matmul(p, w, b):
    _, m, k = p.shape
    n = w.shape[1]
    return pl.pallas_call(
        _final_body,
        grid=(m // _M_BLK,),
        in_specs=[
            pl.BlockSpec((_NC, _M_BLK, k), lambda i: (0, i, 0)),
            pl.BlockSpec((k, n), lambda i: (0, 0)),
            pl.BlockSpec((1, n), lambda i: (0, 0)),
        ],
        out_specs=pl.BlockSpec((_M_BLK, n), lambda i: (i, 0)),
        out_shape=jax.ShapeDtypeStruct((m, n), jnp.float32),
    )(p, w, b)


def kernel(features, edge_index, W0, b0, W1, b1, W2, b2):
    src = edge_index[0].reshape(_NW, _NCHUNK, _CHUNK)
    dst = edge_index[1].reshape(_NW, _NCHUNK, _CHUNK)
    z128 = jnp.zeros((_RPT, 128), jnp.float32)

    t = _matmul(features, W0)
    p = _AGG128(t, src, dst, z128).reshape(_NC, _N, 128)
    t = _fused_matmul(p, b0.reshape(1, -1), W1)
    p = _AGG128(t, src, dst, z128).reshape(_NC, _N, 128)
    t = _relu_add(p, b1.reshape(1, -1))
    p = _AGG128(t, src, dst, z128).reshape(_NC, _N, 128)
    return _final_matmul(p, W2, b2.reshape(1, -1))
